# Initial kernel scaffold; baseline (speedup 1.0000x reference)
import jax
import jax.numpy as jnp
from jax.experimental import pallas as pl


def kernel(obs, conv_w_0, conv_b_0, conv_w_1, conv_b_1, conv_w_2, conv_b_2, mlp_w_0, mlp_b_0, mlp_w_1, mlp_b_1):
    raise NotImplementedError("write your pallas kernel here")


# speedup vs baseline: 30.5726x; 30.5726x over previous
import jax
import jax.numpy as jnp
from jax.experimental import pallas as pl
from jax.experimental.pallas import tpu as pltpu

_TAPS = ((0, 0), (0, 1), (1, 0), (1, 1))


def _conv_as_dense(cw, hin, win, channel_major_in):
    ho, wo = hin - 1, win - 1
    acc = None
    for t, (dh, dw) in enumerate(_TAPS):
        eh = jnp.eye(hin, ho, -dh, dtype=cw.dtype)
        ew = jnp.eye(win, wo, -dw, dtype=cw.dtype)
        spec = 'ij,kl,cn->cikjln' if channel_major_in else 'ij,kl,cn->ikcjln'
        term = jnp.einsum(spec, eh, ew, cw[t])
        acc = term if acc is None else acc + term
    return acc.reshape(hin * win * cw.shape[1], ho * wo * cw.shape[2])


def _fused_body(x_ref, w1_ref, b1_ref, w2_ref, b2_ref, w3_ref, b3_ref,
                w4_ref, b4_ref, w5_ref, b5_ref, o_ref):
    h = x_ref[...]
    h = jnp.maximum(
        jnp.dot(h, w1_ref[...], preferred_element_type=jnp.float32)
        + b1_ref[...], 0.0)
    h = jnp.maximum(
        jnp.dot(h, w2_ref[...], preferred_element_type=jnp.float32)
        + b2_ref[...], 0.0)
    h = jnp.maximum(
        jnp.dot(h, w3_ref[...], preferred_element_type=jnp.float32)
        + b3_ref[...], 0.0)
    h = jnp.maximum(
        jnp.dot(h, w4_ref[...], preferred_element_type=jnp.float32)
        + b4_ref[...], 0.0)
    o_ref[...] = (jnp.dot(h, w5_ref[...], preferred_element_type=jnp.float32)
                  + b5_ref[...]).astype(o_ref.dtype)


def kernel(obs, conv_w_0, conv_b_0, conv_w_1, conv_b_1, conv_w_2, conv_b_2,
           mlp_w_0, mlp_b_0, mlp_w_1, mlp_b_1):
    B, cin, H, W = obs.shape
    h1, w1s = H - 1, W - 1
    h2, w2s = h1 - 1, w1s - 1
    h3, w3s = h2 - 1, w2s - 1
    c1, c2, c3 = conv_w_0.shape[2], conv_w_1.shape[2], conv_w_2.shape[2]
    hid = mlp_w_0.shape[-1]
    na = mlp_w_1.shape[-1]

    dw1 = _conv_as_dense(conv_w_0, H, W, True)
    dw2 = _conv_as_dense(conv_w_1, h1, w1s, False)
    dw3 = _conv_as_dense(conv_w_2, h2, w2s, False)
    dw4 = mlp_w_0.reshape(h3 * w3s * c3, hid)
    dw5 = mlp_w_1
    db1 = jnp.tile(conv_b_0, (1, h1 * w1s))
    db2 = jnp.tile(conv_b_1, (1, h2 * w2s))
    db3 = jnp.tile(conv_b_2, (1, h3 * w3s))

    x2d = obs.reshape(B, cin * H * W)

    bt = min(B, 2048)
    b_pad = pl.cdiv(B, bt) * bt
    if b_pad != B:
        x2d = jnp.pad(x2d, ((0, b_pad - B), (0, 0)))
    steps = b_pad // bt

    k1 = cin * H * W
    ws = [dw1, db1, dw2, db2, dw3, db3, dw4, mlp_b_0, dw5, mlp_b_1]
    in_specs = [pl.BlockSpec((bt, k1), lambda i: (i, 0))]
    in_specs += [pl.BlockSpec(w.shape, lambda i: (0, 0)) for w in ws]

    sizes = [(k1, h1 * w1s * c1), (h1 * w1s * c1, h2 * w2s * c2),
             (h2 * w2s * c2, h3 * w3s * c3), (h3 * w3s * c3, hid), (hid, na)]
    flops = 2 * b_pad * sum(a * b for a, b in sizes)
    nbytes = 4 * (x2d.size + sum(w.size for w in ws) + b_pad * na)

    out = pl.pallas_call(
        _fused_body,
        out_shape=jax.ShapeDtypeStruct((b_pad, na), jnp.float32),
        grid=(steps,),
        in_specs=in_specs,
        out_specs=pl.BlockSpec((bt, na), lambda i: (i, 0)),
        compiler_params=pltpu.CompilerParams(
            dimension_semantics=("parallel",)),
        cost_estimate=pl.CostEstimate(
            flops=int(flops), transcendentals=0, bytes_accessed=int(nbytes)),
    )(x2d, *ws)
    return out[:B]
